# baseline (speedup 1.0000x reference)
import functools

import jax
import jax.numpy as jnp
from jax.experimental import pallas as pl

TOP_K = 8
NEG_BIG = -1e30


def _gate_kernel(x_ref, w0_ref, b0_ref, w1_ref, b1_ref, out_ref, *, hw):
    xt = x_ref[...]
    pooled = jnp.max(xt, axis=2) + jnp.sum(xt, axis=2) * (1.0 / hw)

    dn = (((1,), (1,)), ((), ()))
    h = jax.lax.dot_general(pooled, w1_ref[...], dn,
                            preferred_element_type=jnp.float32) + b1_ref[...]
    h = jnp.where(h >= 0, h, 0.2 * h)

    z = jax.lax.dot_general(pooled, w0_ref[...], dn,
                            preferred_element_type=jnp.float32) + b0_ref[...]
    noise = jnp.maximum(z, 0.0) + jnp.log1p(jnp.exp(-jnp.abs(z)))

    e = noise.shape[1]
    nmean = jnp.mean(noise, axis=1, keepdims=True)
    d = noise - nmean
    var = jnp.sum(d * d, axis=1, keepdims=True) * (1.0 / (e - 1))
    std = jnp.sqrt(var)
    std = jnp.where(std == 0, 1.0, std)
    scores = h + d / std

    iota = jax.lax.broadcasted_iota(jnp.int32, scores.shape, 1)
    work = scores
    mask = jnp.zeros_like(scores, dtype=jnp.bool_)
    for _ in range(TOP_K):
        m = jnp.max(work, axis=1, keepdims=True)
        first = jnp.min(jnp.where(work == m, iota, e), axis=1, keepdims=True)
        sel = iota == first
        mask = jnp.logical_or(mask, sel)
        work = jnp.where(sel, NEG_BIG, work)

    h_masked = jnp.where(mask, h, -1e9)
    hm = jnp.max(h_masked, axis=1, keepdims=True)
    ex = jnp.exp(h_masked - hm)
    out_ref[...] = ex / jnp.sum(ex, axis=1, keepdims=True)


@functools.partial(jax.jit, static_argnames=("interpret",))
def kernel(x, W0, b0, W1, b1, interpret=False):
    B, C, H, W = x.shape
    E = W0.shape[0]
    hw = H * W
    x3 = x.reshape(B, C, hw)
    bt = 16
    grid = (B // bt,)
    out = pl.pallas_call(
        functools.partial(_gate_kernel, hw=float(hw)),
        grid=grid,
        in_specs=[
            pl.BlockSpec((bt, C, hw), lambda i: (i, 0, 0)),
            pl.BlockSpec((E, C), lambda i: (0, 0)),
            pl.BlockSpec((1, E), lambda i: (0, 0)),
            pl.BlockSpec((E, C), lambda i: (0, 0)),
            pl.BlockSpec((1, E), lambda i: (0, 0)),
        ],
        out_specs=pl.BlockSpec((bt, E), lambda i: (i, 0)),
        out_shape=jax.ShapeDtypeStruct((B, E), jnp.float32),
        interpret=interpret,
    )(x3, W0, b0.reshape(1, E), W1, b1.reshape(1, E))
    return out
